# Initial kernel scaffold; baseline (speedup 1.0000x reference)
import jax
import jax.numpy as jnp
from jax.experimental import pallas as pl


def kernel(h, edge_index, edge_weight, W0, b0, W1, b1, W2, b2):
    raise NotImplementedError("write your pallas kernel here")


# speedup vs baseline: 5.2035x; 5.2035x over previous
import functools

import jax
import jax.numpy as jnp
from jax import lax
from jax.experimental import pallas as pl
from jax.experimental.pallas import tpu as pltpu
from jax.experimental.pallas import tpu_sc as plsc

NN = 10000
NE = 320000
D = 128
NC = 2
NS = 16
NW = NC * NS
E_PER_W = NE // NW
CHUNK = 80
NCHUNK = E_PER_W // CHUNK
OUT_TILES = 10
ROWS_PER_OTILE = NN // OUT_TILES
ZROWS = 40


def _bcast0(v):
    return lax.gather(
        v, jnp.zeros((16, 1), jnp.int32),
        lax.GatherDimensionNumbers(
            offset_dims=(), collapsed_slice_dims=(0,), start_index_map=(0,)),
        slice_sizes=(1,),
        mode=lax.GatherScatterMode.PROMISE_IN_BOUNDS)


def _sc_body(h_hbm, src_hbm, dst_hbm, w_hbm, out_hbm,
             src_l, dst_l, w_l, rows, agg, sem, sem2):
    c = lax.axis_index("c")
    s = lax.axis_index("s")
    wid = c * NS + s

    pltpu.sync_copy(src_hbm.at[wid, 0], src_l)
    pltpu.sync_copy(dst_hbm.at[wid], dst_l)

    @pl.when(s < OUT_TILES)
    def _zero():
        def zfill(i, carry):
            for j in range(D // 16):
                rows[i, pl.ds(j * 16, 16)] = jnp.zeros((16,), jnp.float32)
            return carry
        lax.fori_loop(0, ZROWS, zfill, 0)
        for k in range(ROWS_PER_OTILE // ZROWS):
            pltpu.sync_copy(
                rows.at[pl.ds(0, ZROWS)],
                agg.at[pl.ds(s * ROWS_PER_OTILE + k * ZROWS, ZROWS)])
    plsc.subcore_barrier()

    def chunk_body(i, carry):
        cw = pltpu.async_copy(w_hbm.at[wid, i], w_l.at[pl.ds(0, CHUNK)], sem2)
        cg = pltpu.async_copy(
            h_hbm.at[src_l.at[pl.ds(i * CHUNK, CHUNK)]], rows, sem)
        cw.wait()
        cg.wait()

        def edge(e, ecarry):
            wv = _bcast0(w_l[pl.ds(e, 16)])
            for j in range(D // 16):
                sl = pl.ds(j * 16, 16)
                rows[e, sl] = rows[e, sl] * wv
            return ecarry
        lax.fori_loop(0, CHUNK, edge, 0)

        pltpu.sync_copy(rows, agg.at[dst_l.at[i]], add=True)
        return carry
    lax.fori_loop(0, NCHUNK, chunk_body, 0)

    plsc.subcore_barrier()

    @pl.when(s < OUT_TILES)
    def _copy_out():
        pltpu.sync_copy(
            agg.at[pl.ds(s * ROWS_PER_OTILE, ROWS_PER_OTILE)],
            out_hbm.at[pl.ds(c * NN + s * ROWS_PER_OTILE, ROWS_PER_OTILE)])


_sc_aggregate = functools.partial(
    pl.kernel,
    _sc_body,
    out_type=jax.ShapeDtypeStruct((NC * NN, D), jnp.float32),
    mesh=plsc.VectorSubcoreMesh(core_axis_name="c", subcore_axis_name="s"),
    scratch_types=[
        pltpu.VMEM((E_PER_W,), jnp.int32),
        pltpu.VMEM((NCHUNK, CHUNK), jnp.int32),
        pltpu.VMEM((CHUNK + 48,), jnp.float32),
        pltpu.VMEM((CHUNK, D), jnp.float32),
        pltpu.VMEM_SHARED((NN, D), jnp.float32),
        pltpu.SemaphoreType.DMA,
        pltpu.SemaphoreType.DMA,
    ],
    compiler_params=pltpu.CompilerParams(use_tc_tiling_on_sc=False),
)()


def _mm_body(p0_ref, p1_ref, w_ref, b_ref, o_ref, *, act):
    x = p0_ref[...] + p1_ref[...]
    y = jnp.dot(x, w_ref[...], preferred_element_type=jnp.float32) + b_ref[...]
    o_ref[...] = jnp.tanh(y) if act else y


def _tc_layer(p0, p1, W, b, act):
    R = 2000
    return pl.pallas_call(
        functools.partial(_mm_body, act=act),
        grid=(NN // R,),
        in_specs=[
            pl.BlockSpec((R, D), lambda i: (i, 0)),
            pl.BlockSpec((R, D), lambda i: (i, 0)),
            pl.BlockSpec((D, D), lambda i: (0, 0)),
            pl.BlockSpec((1, D), lambda i: (0, 0)),
        ],
        out_specs=pl.BlockSpec((R, D), lambda i: (i, 0)),
        out_shape=jax.ShapeDtypeStruct((NN, D), jnp.float32),
    )(p0, p1, W, b.reshape(1, D))


def kernel(h, edge_index, edge_weight, W0, b0, W1, b1, W2, b2):
    src3 = edge_index[0].astype(jnp.int32).reshape(NW, 1, E_PER_W)
    dst3 = edge_index[1].astype(jnp.int32).reshape(NW, NCHUNK, CHUNK)
    w3 = edge_weight.astype(jnp.float32).reshape(NW, NCHUNK, CHUNK)
    layers = [(W0, b0), (W1, b1), (W2, b2)]
    outs = [h]
    cur = h
    for l, (W, b) in enumerate(layers):
        part = _sc_aggregate(cur, src3, dst3, w3)
        cur = _tc_layer(part[:NN], part[NN:], W, b, act=(l < 2))
        outs.append(cur)
    return jnp.concatenate(outs, axis=1)


# speedup vs baseline: 10.3600x; 1.9910x over previous
import functools

import jax
import jax.numpy as jnp
from jax import lax
from jax.experimental import pallas as pl
from jax.experimental.pallas import tpu as pltpu
from jax.experimental.pallas import tpu_sc as plsc

NN = 10000
NE = 320000
D = 128
NC = 2
NS = 16
NW = NC * NS
E_PER_W = NE // NW
CHUNK = 80
NCHUNK = E_PER_W // CHUNK
OUT_TILES = 10
ROWS_PER_OTILE = NN // OUT_TILES
ZROWS = 40


def _bcast_lane(v, k):
    return lax.gather(
        v, jnp.full((16, 1), k, jnp.int32),
        lax.GatherDimensionNumbers(
            offset_dims=(), collapsed_slice_dims=(0,), start_index_map=(0,)),
        slice_sizes=(1,),
        mode=lax.GatherScatterMode.PROMISE_IN_BOUNDS)


def _sc_body(h_hbm, src_hbm, dst_hbm, w_hbm, out_hbm,
             src_l, dst_l, w_l, rows0, rows1, agg, sem0, sem1):
    c = lax.axis_index("c")
    s = lax.axis_index("s")
    wid = c * NS + s

    pltpu.sync_copy(src_hbm.at[wid, 0], src_l)
    pltpu.sync_copy(dst_hbm.at[wid], dst_l)
    pltpu.sync_copy(w_hbm.at[wid, 0], w_l)

    @pl.when(s < OUT_TILES)
    def _zero():
        def zfill(i, carry):
            for j in range(D // 16):
                rows0[i, pl.ds(j * 16, 16)] = jnp.zeros((16,), jnp.float32)
            return carry
        lax.fori_loop(0, ZROWS, zfill, 0)
        for k in range(ROWS_PER_OTILE // ZROWS):
            pltpu.sync_copy(
                rows0.at[pl.ds(0, ZROWS)],
                agg.at[pl.ds(s * ROWS_PER_OTILE + k * ZROWS, ZROWS)])
    plsc.subcore_barrier()

    bufs = (rows0, rows1)
    sems = (sem0, sem1)

    pltpu.async_copy(h_hbm.at[src_l.at[pl.ds(0, CHUNK)]], rows0, sem0)

    def chunk_body(i, carry):
        for p in range(2):
            @pl.when((i % 2) == p)
            def _do(p=p):
                cur, nxt = bufs[p], bufs[1 - p]

                @pl.when(i + 1 < NCHUNK)
                def _prefetch():
                    pltpu.async_copy(
                        h_hbm.at[src_l.at[pl.ds((i + 1) * CHUNK, CHUNK)]],
                        nxt, sems[1 - p])

                pltpu.make_async_copy(
                    h_hbm.at[src_l.at[pl.ds(0, CHUNK)]], cur, sems[p]).wait()

                def group(g, gcarry):
                    wv16 = w_l[pl.ds(i * CHUNK + g * 16, 16)]
                    for k in range(16):
                        wv = _bcast_lane(wv16, k)
                        e = g * 16 + k
                        for j in range(D // 16):
                            sl = pl.ds(j * 16, 16)
                            cur[e, sl] = cur[e, sl] * wv
                    return gcarry
                lax.fori_loop(0, CHUNK // 16, group, 0)

                pltpu.sync_copy(cur, agg.at[dst_l.at[i]], add=True)
        return carry
    lax.fori_loop(0, NCHUNK, chunk_body, 0)

    plsc.subcore_barrier()

    @pl.when(s < OUT_TILES)
    def _copy_out():
        pltpu.sync_copy(
            agg.at[pl.ds(s * ROWS_PER_OTILE, ROWS_PER_OTILE)],
            out_hbm.at[pl.ds(c * NN + s * ROWS_PER_OTILE, ROWS_PER_OTILE)])


_sc_aggregate = functools.partial(
    pl.kernel,
    _sc_body,
    out_type=jax.ShapeDtypeStruct((NC * NN, D), jnp.float32),
    mesh=plsc.VectorSubcoreMesh(core_axis_name="c", subcore_axis_name="s"),
    scratch_types=[
        pltpu.VMEM((E_PER_W,), jnp.int32),
        pltpu.VMEM((NCHUNK, CHUNK), jnp.int32),
        pltpu.VMEM((E_PER_W,), jnp.float32),
        pltpu.VMEM((CHUNK, D), jnp.float32),
        pltpu.VMEM((CHUNK, D), jnp.float32),
        pltpu.VMEM_SHARED((NN, D), jnp.float32),
        pltpu.SemaphoreType.DMA,
        pltpu.SemaphoreType.DMA,
    ],
    compiler_params=pltpu.CompilerParams(use_tc_tiling_on_sc=False),
)()


def _mm_body(p0_ref, p1_ref, w_ref, b_ref, o_ref, *, act):
    x = p0_ref[...] + p1_ref[...]
    y = jnp.dot(x, w_ref[...], preferred_element_type=jnp.float32) + b_ref[...]
    o_ref[...] = jnp.tanh(y) if act else y


def _tc_layer(p0, p1, W, b, act):
    R = 2000
    return pl.pallas_call(
        functools.partial(_mm_body, act=act),
        grid=(NN // R,),
        in_specs=[
            pl.BlockSpec((R, D), lambda i: (i, 0)),
            pl.BlockSpec((R, D), lambda i: (i, 0)),
            pl.BlockSpec((D, D), lambda i: (0, 0)),
            pl.BlockSpec((1, D), lambda i: (0, 0)),
        ],
        out_specs=pl.BlockSpec((R, D), lambda i: (i, 0)),
        out_shape=jax.ShapeDtypeStruct((NN, D), jnp.float32),
    )(p0, p1, W, b.reshape(1, D))


def kernel(h, edge_index, edge_weight, W0, b0, W1, b1, W2, b2):
    src3 = edge_index[0].astype(jnp.int32).reshape(NW, 1, E_PER_W)
    dst3 = edge_index[1].astype(jnp.int32).reshape(NW, NCHUNK, CHUNK)
    w3 = edge_weight.astype(jnp.float32).reshape(NW, 1, E_PER_W)
    layers = [(W0, b0), (W1, b1), (W2, b2)]
    outs = [h]
    cur = h
    for l, (W, b) in enumerate(layers):
        part = _sc_aggregate(cur, src3, dst3, w3)
        cur = _tc_layer(part[:NN], part[NN:], W, b, act=(l < 2))
        outs.append(cur)
    return jnp.concatenate(outs, axis=1)
